# baseline (speedup 1.0000x reference)
import functools

import jax
import jax.numpy as jnp
from jax import lax
from jax.experimental import pallas as pl
from jax.experimental.pallas import tpu as pltpu
from jax.experimental.pallas import tpu_sc as plsc

_E = 8
_D = 1024
_H = 1024
_RHO = 0.5
_N = 2048
_TG = 256
_NTILE = 24
_P = _NTILE * _TG
_ZP = _P - 1
_NW = 32
_CH = 64


def _router_kernel(x_ref, gwt_ref, bias_ref, vnull_ref,
                   meta_ref, cnt_ref, aux_ref, a1_ref, a2_ref, cw_ref):
    x = x_ref[...]
    gwt = gwt_ref[...]
    l = jnp.dot(x, gwt, preferred_element_type=jnp.float32) + bias_ref[...]
    v = vnull_ref[0, 0]

    idx = lax.broadcasted_iota(jnp.int32, (_N, _E), 1)
    l1 = jnp.max(l, axis=-1, keepdims=True)
    i1 = jnp.min(jnp.where(l == l1, idx, _E), axis=-1, keepdims=True)
    oh1 = idx == i1
    s1_real = l1 >= v
    lm = jnp.where(oh1, -jnp.inf, l)
    l2 = jnp.max(lm, axis=-1, keepdims=True)
    i2 = jnp.min(jnp.where(lm == l2, idx, _E), axis=-1, keepdims=True)
    oh2 = idx == i2
    s2_real = s1_real & (l2 >= v)

    m = jnp.maximum(l1, v)
    el = jnp.exp(l - m)
    ev = jnp.exp(v - m)
    z = jnp.sum(el, axis=-1, keepdims=True) + 8.0 * ev
    p = el / z
    w1 = jnp.where(s1_real, jnp.sum(jnp.where(oh1, p, 0.0), axis=-1, keepdims=True), 0.0)
    w2 = jnp.where(s2_real, jnp.sum(jnp.where(oh2, p, 0.0), axis=-1, keepdims=True), 0.0)
    wsum = jnp.maximum(w1 + w2, 1e-6)
    w1n = w1 / wsum
    w2n = w2 / wsum

    elr = jnp.exp(l - l1)
    pr = elr / jnp.sum(elr, axis=-1, keepdims=True)
    p_real = jnp.mean(pr, axis=0)
    a1 = (oh1 & s1_real).astype(jnp.float32)
    a2 = (oh2 & s2_real).astype(jnp.float32)
    counts = jnp.sum(a1 + a2, axis=0)
    total = jnp.maximum(jnp.sum(counts), 1e-6)
    l_bal = _E * jnp.sum((counts / total) * p_real)
    n_real = jnp.sum(a1) + jnp.sum(a2)
    null_rate = (2.0 * _N - n_real) / (2.0 * _N)
    l_null = (null_rate - _RHO) ** 2
    lse = m + jnp.log(z)
    l_z = jnp.mean(lse * lse)
    aux = 0.02 * l_bal + 0.001 * l_z + 0.01 * l_null
    aux_ref[...] = jnp.reshape(aux, (1, 1))

    cnt1 = jnp.sum(a1, axis=0, keepdims=True)
    cnt2 = jnp.sum(a2, axis=0, keepdims=True)
    cnt = cnt1 + cnt2
    pc = jnp.ceil(cnt * (1.0 / _TG)) * _TG
    eidx = lax.broadcasted_iota(jnp.int32, (_E, _E), 0)
    ejdx = lax.broadcasted_iota(jnp.int32, (_E, _E), 1)
    strict_lt = (eidx < ejdx).astype(jnp.float32)
    base = jnp.dot(pc, strict_lt, preferred_element_type=jnp.float32)
    base2 = base + cnt1
    cnt_ref[...] = jnp.concatenate(
        [base, pc, jnp.zeros((1, 16), jnp.float32)], axis=1)

    a1_ref[...] = a1
    a2_ref[...] = a2
    cw_ref[...] = jnp.concatenate(
        [w1n, w2n, s1_real.astype(jnp.float32), s2_real.astype(jnp.float32),
         jnp.zeros((_N, 4), jnp.float32)], axis=1)

    ck = _N // 8
    rower = lax.broadcasted_iota(jnp.int32, (ck, ck), 0)
    coler = lax.broadcasted_iota(jnp.int32, (ck, ck), 1)
    tri = (coler < rower).astype(jnp.float32)

    def chunk(k, carry):
        carry1, carry2 = carry
        sl = pl.ds(k * ck, ck)
        a1c = a1_ref[sl, :]
        a2c = a2_ref[sl, :]
        cc = cw_ref[sl, :]
        w1c = cc[:, 0:1]
        w2c = cc[:, 1:2]
        s1c = cc[:, 2:3] > 0.5
        s2c = cc[:, 3:4] > 0.5
        r1c = jnp.dot(tri, a1c, preferred_element_type=jnp.float32) + carry1
        r2c = jnp.dot(tri, a2c, preferred_element_type=jnp.float32) + carry2
        d1 = jnp.sum(a1c * (base + r1c), axis=-1, keepdims=True)
        d2 = jnp.sum(a2c * (base2 + r2c), axis=-1, keepdims=True)
        d1c = jnp.where(s1c, d1, float(_ZP))
        d2c = jnp.where(s2c, d2, float(_ZP))
        dest1 = jnp.where(s1c, d1, -1.0)
        dest2 = jnp.where(s2c, d2, -1.0)
        meta_ref[sl, :] = jnp.concatenate(
            [w1c, w2c, d1c, d2c, dest1, dest2,
             jnp.zeros((ck, 2), jnp.float32)], axis=1)
        return (carry1 + jnp.sum(a1c, axis=0, keepdims=True),
                carry2 + jnp.sum(a2c, axis=0, keepdims=True))

    lax.fori_loop(0, 8, chunk, (jnp.zeros((1, _E), jnp.float32),
                                jnp.zeros((1, _E), jnp.float32)))


def _router(xf, gate_W, logit_bias, null_logit):
    return pl.pallas_call(
        _router_kernel,
        out_shape=(
            jax.ShapeDtypeStruct((_N, 8), jnp.float32),
            jax.ShapeDtypeStruct((1, 32), jnp.float32),
            jax.ShapeDtypeStruct((1, 1), jnp.float32),
        ),
        scratch_shapes=[
            pltpu.VMEM((_N, _E), jnp.float32),
            pltpu.VMEM((_N, _E), jnp.float32),
            pltpu.VMEM((_N, 8), jnp.float32),
        ],
    )(xf, gate_W.T, logit_bias.reshape(1, _E), null_logit.reshape(1, 1))


def _sc_scatter(destcat, wcat):
    mesh = plsc.VectorSubcoreMesh(core_axis_name="c", subcore_axis_name="s")

    @functools.partial(
        pl.kernel, mesh=mesh,
        compiler_params=pltpu.CompilerParams(needs_layout_passes=False),
        out_type=(jax.ShapeDtypeStruct((_P,), jnp.int32),
                  jax.ShapeDtypeStruct((_P,), jnp.float32)),
        scratch_types=[
            pltpu.VMEM((2 * _N,), jnp.int32),
            pltpu.VMEM((2 * _N,), jnp.float32),
            pltpu.VMEM((_P,), jnp.int32),
            pltpu.VMEM((_P,), jnp.float32),
        ],
    )
    def k(dest_hbm, w_hbm, src_out, wsrt_out, dest_v, w_v, src_v, wsrt_v):
        wid = lax.axis_index("s") * 2 + lax.axis_index("c")

        @pl.when(wid == 0)
        def _():
            pltpu.sync_copy(dest_hbm, dest_v)
            pltpu.sync_copy(w_hbm, w_v)
            zi = jnp.zeros((16,), jnp.int32)
            zf = jnp.zeros((16,), jnp.float32)

            def zbody(i, c):
                src_v[pl.ds(i * 16, 16)] = zi
                wsrt_v[pl.ds(i * 16, 16)] = zf
                return c

            lax.fori_loop(0, _P // 16, zbody, 0)
            iot = lax.iota(jnp.int32, 16)

            def body(i, c):
                d = dest_v[pl.ds(i * 16, 16)]
                wv = w_v[pl.ds(i * 16, 16)]
                msk = d >= 0
                ds = jnp.where(msk, d, 0)
                tok = (i % 128) * 16 + iot
                plsc.store_scatter(src_v, [ds], tok, mask=msk)
                plsc.store_scatter(wsrt_v, [ds], wv, mask=msk)
                return c

            lax.fori_loop(0, (2 * _N) // 16, body, 0)
            pltpu.sync_copy(src_v, src_out)
            pltpu.sync_copy(wsrt_v, wsrt_out)

    return k(destcat, wcat)


def _sc_gather(table, idx, nrows, chunk):
    chunks = nrows // (_NW * chunk)
    idx3 = idx.reshape(_NW, chunks, chunk)
    hw = _D // 2
    mesh = plsc.VectorSubcoreMesh(core_axis_name="c", subcore_axis_name="s")

    @functools.partial(
        pl.kernel, mesh=mesh,
        compiler_params=pltpu.CompilerParams(needs_layout_passes=False),
        out_type=jax.ShapeDtypeStruct((nrows, hw), jnp.int32),
        scratch_types=[
            pltpu.VMEM((chunks, chunk), jnp.int32),
            [pltpu.VMEM((chunk, hw), jnp.int32) for _ in range(chunks)],
            pltpu.SemaphoreType.DMA,
        ],
    )
    def k(table_hbm, idx_hbm, out_hbm, idx_v, rows_v, sem):
        wid = lax.axis_index("s") * 2 + lax.axis_index("c")
        base = wid * (chunks * chunk)
        pltpu.sync_copy(idx_hbm.at[wid], idx_v)
        cps = [pltpu.async_copy(table_hbm.at[idx_v.at[c]], rows_v[c], sem)
               for c in range(chunks)]
        for c in range(chunks):
            cps[c].wait()
            pltpu.sync_copy(rows_v[c], out_hbm.at[pl.ds(base + c * chunk, chunk)])

    return k(table, idx3)


def _gmm_kernel(emap_ref, tval_ref, xs_ref, wg_ref, wu_ref, wd_ref, ws_ref,
                out_ref, wgb_ref, wub_ref, wdb_ref):
    j = pl.program_id(0)
    fresh = jnp.logical_or(j == 0, emap_ref[j] != emap_ref[jnp.maximum(j - 1, 0)])

    @pl.when(jnp.logical_and(tval_ref[j] == 1, fresh))
    def _():
        wgb_ref[...] = wg_ref[0].astype(jnp.bfloat16)
        wub_ref[...] = wu_ref[0].astype(jnp.bfloat16)
        wdb_ref[...] = wd_ref[0].astype(jnp.bfloat16)

    @pl.when(tval_ref[j] == 1)
    def _():
        xb = xs_ref[...]
        g = jnp.dot(xb, wgb_ref[...], preferred_element_type=jnp.float32)
        u = jnp.dot(xb, wub_ref[...], preferred_element_type=jnp.float32)
        h = (g * jax.nn.sigmoid(g) * u).astype(jnp.bfloat16)
        y = jnp.dot(h, wdb_ref[...], preferred_element_type=jnp.float32)
        out_ref[...] = (ws_ref[0, 0, :][:, None] * y).astype(jnp.bfloat16)

    @pl.when(tval_ref[j] == 0)
    def _():
        out_ref[...] = jnp.zeros((_TG, _D), jnp.bfloat16)


def _grouped_mm(xs, W_gate, W_up, W_down, wsrt, emap, tval):
    return pl.pallas_call(
        _gmm_kernel,
        grid_spec=pltpu.PrefetchScalarGridSpec(
            num_scalar_prefetch=2,
            grid=(_NTILE,),
            in_specs=[
                pl.BlockSpec((_TG, _D), lambda j, em, tv: (j, 0)),
                pl.BlockSpec((1, _D, _H), lambda j, em, tv: (em[j], 0, 0)),
                pl.BlockSpec((1, _D, _H), lambda j, em, tv: (em[j], 0, 0)),
                pl.BlockSpec((1, _H, _D), lambda j, em, tv: (em[j], 0, 0)),
                pl.BlockSpec((1, 1, _TG), lambda j, em, tv: (j, 0, 0)),
            ],
            out_specs=pl.BlockSpec((_TG, _D), lambda j, em, tv: (j, 0)),
            scratch_shapes=[
                pltpu.VMEM((_D, _H), jnp.bfloat16),
                pltpu.VMEM((_D, _H), jnp.bfloat16),
                pltpu.VMEM((_H, _D), jnp.bfloat16),
            ],
        ),
        out_shape=jax.ShapeDtypeStruct((_P, _D), jnp.bfloat16),
        compiler_params=pltpu.CompilerParams(
            dimension_semantics=("arbitrary",),
        ),
    )(emap, tval, xs, W_gate, W_up, W_down, wsrt)


def _shared_kernel(x_ref, sg_ref, su_ref, sd_ref, r1_ref, r2_ref, out_ref,
                   sgb_ref, sub_ref, sdb_ref):
    @pl.when(pl.program_id(0) == 0)
    def _():
        sgb_ref[...] = sg_ref[...].astype(jnp.bfloat16)
        sub_ref[...] = su_ref[...].astype(jnp.bfloat16)
        sdb_ref[...] = sd_ref[...].astype(jnp.bfloat16)

    xb = x_ref[...].astype(jnp.bfloat16)
    dn = (((1,), (1,)), ((), ()))
    g = lax.dot_general(xb, sgb_ref[...], dn, preferred_element_type=jnp.float32)
    u = lax.dot_general(xb, sub_ref[...], dn, preferred_element_type=jnp.float32)
    h = (g * jax.nn.sigmoid(g) * u).astype(jnp.bfloat16)
    sh = lax.dot_general(h, sdb_ref[...], dn, preferred_element_type=jnp.float32)
    out_ref[...] = (sh + r1_ref[...].astype(jnp.float32)
                    + r2_ref[...].astype(jnp.float32))


def _shared_combine(xf, sg_W, su_W, sd_W, rcat):
    nt = _N // _TG
    return pl.pallas_call(
        _shared_kernel,
        grid=(nt,),
        in_specs=[
            pl.BlockSpec((_TG, _D), lambda j: (j, 0)),
            pl.BlockSpec((_H, _D), lambda j: (0, 0)),
            pl.BlockSpec((_H, _D), lambda j: (0, 0)),
            pl.BlockSpec((_D, _H), lambda j: (0, 0)),
            pl.BlockSpec((_TG, _D), lambda j: (j, 0)),
            pl.BlockSpec((_TG, _D), lambda j: (j + nt, 0)),
        ],
        out_specs=pl.BlockSpec((_TG, _D), lambda j: (j, 0)),
        out_shape=jax.ShapeDtypeStruct((_N, _D), jnp.float32),
        scratch_shapes=[
            pltpu.VMEM((_H, _D), jnp.bfloat16),
            pltpu.VMEM((_H, _D), jnp.bfloat16),
            pltpu.VMEM((_D, _H), jnp.bfloat16),
        ],
        compiler_params=pltpu.CompilerParams(
            dimension_semantics=("arbitrary",),
        ),
    )(xf, sg_W, su_W, sd_W, rcat, rcat)


def kernel(x, gate_W, logit_bias, null_logit, W_gate, W_up, W_down, sg_W, su_W, sd_W):
    b, t, d = x.shape
    xf = x.reshape(_N, _D)

    meta, cnts, aux = _router(xf, gate_W, logit_bias, null_logit)

    base = cnts[0, :_E]
    pc = cnts[0, _E:2 * _E]
    ends = base + pc
    jpos = jnp.arange(_NTILE, dtype=jnp.float32) * _TG
    emap = jnp.minimum(
        jnp.sum((jpos[:, None] >= ends[None, :]).astype(jnp.int32), axis=1),
        _E - 1).astype(jnp.int32)
    tval = (jpos < jnp.sum(pc)).astype(jnp.int32)

    destcat = jnp.concatenate([meta[:, 4], meta[:, 5]]).astype(jnp.int32)
    wcat = jnp.concatenate([meta[:, 0], meta[:, 1]])
    dcat = jnp.concatenate([meta[:, 2], meta[:, 3]]).astype(jnp.int32)

    src, wsrt = _sc_scatter(destcat, wcat)
    xb32 = lax.bitcast_convert_type(
        xf.astype(jnp.bfloat16).reshape(_N, _D // 2, 2), jnp.int32)
    xs32 = _sc_gather(xb32, src, _P, 96)
    xs = lax.bitcast_convert_type(
        xs32.reshape(_P, _D // 2, 1), jnp.bfloat16).reshape(_P, _D)
    ys = _grouped_mm(xs, W_gate, W_up, W_down, wsrt.reshape(_NTILE, 1, _TG),
                     emap, tval)
    ys32 = lax.bitcast_convert_type(ys.reshape(_P, _D // 2, 2), jnp.int32)
    r32 = _sc_gather(ys32, dcat, 2 * _N, 128)
    rcat = lax.bitcast_convert_type(
        r32.reshape(2 * _N, _D // 2, 1), jnp.bfloat16).reshape(2 * _N, _D)
    out = _shared_combine(xf, sg_W, su_W, sd_W, rcat)

    return out.reshape(b, t, d), aux[0, 0]


# speedup vs baseline: 3.5556x; 3.5556x over previous
import functools

import jax
import jax.numpy as jnp
from jax import lax
from jax.experimental import pallas as pl
from jax.experimental.pallas import tpu as pltpu
from jax.experimental.pallas import tpu_sc as plsc

_E = 8
_D = 1024
_H = 1024
_RHO = 0.5
_N = 2048
_TG = 128
_NTILE = 40
_P = _NTILE * _TG
_ZP = _P - 1


def _router_kernel(x_ref, gwt_ref, bias_ref, vnull_ref,
                   meta_ref, cnt_ref, aux_ref, a1_ref, a2_ref, cw_ref):
    x = x_ref[...]
    gwt = gwt_ref[...]
    l = jnp.dot(x, gwt, preferred_element_type=jnp.float32) + bias_ref[...]
    v = vnull_ref[0, 0]

    idx = lax.broadcasted_iota(jnp.int32, (_N, _E), 1)
    l1 = jnp.max(l, axis=-1, keepdims=True)
    i1 = jnp.min(jnp.where(l == l1, idx, _E), axis=-1, keepdims=True)
    oh1 = idx == i1
    s1_real = l1 >= v
    lm = jnp.where(oh1, -jnp.inf, l)
    l2 = jnp.max(lm, axis=-1, keepdims=True)
    i2 = jnp.min(jnp.where(lm == l2, idx, _E), axis=-1, keepdims=True)
    oh2 = idx == i2
    s2_real = s1_real & (l2 >= v)

    m = jnp.maximum(l1, v)
    el = jnp.exp(l - m)
    ev = jnp.exp(v - m)
    z = jnp.sum(el, axis=-1, keepdims=True) + 8.0 * ev
    p = el / z
    w1 = jnp.where(s1_real, jnp.sum(jnp.where(oh1, p, 0.0), axis=-1, keepdims=True), 0.0)
    w2 = jnp.where(s2_real, jnp.sum(jnp.where(oh2, p, 0.0), axis=-1, keepdims=True), 0.0)
    wsum = jnp.maximum(w1 + w2, 1e-6)
    w1n = w1 / wsum
    w2n = w2 / wsum

    elr = jnp.exp(l - l1)
    pr = elr / jnp.sum(elr, axis=-1, keepdims=True)
    p_real = jnp.mean(pr, axis=0)
    a1 = (oh1 & s1_real).astype(jnp.float32)
    a2 = (oh2 & s2_real).astype(jnp.float32)
    counts = jnp.sum(a1 + a2, axis=0)
    total = jnp.maximum(jnp.sum(counts), 1e-6)
    l_bal = _E * jnp.sum((counts / total) * p_real)
    n_real = jnp.sum(a1) + jnp.sum(a2)
    null_rate = (2.0 * _N - n_real) / (2.0 * _N)
    l_null = (null_rate - _RHO) ** 2
    lse = m + jnp.log(z)
    l_z = jnp.mean(lse * lse)
    aux = 0.02 * l_bal + 0.001 * l_z + 0.01 * l_null
    aux_ref[...] = jnp.reshape(aux, (1, 1))

    cnt1 = jnp.sum(a1, axis=0, keepdims=True)
    cnt2 = jnp.sum(a2, axis=0, keepdims=True)
    cnt = cnt1 + cnt2
    pc = jnp.ceil(cnt * (1.0 / _TG)) * _TG
    eidx = lax.broadcasted_iota(jnp.int32, (_E, _E), 0)
    ejdx = lax.broadcasted_iota(jnp.int32, (_E, _E), 1)
    strict_lt = (eidx < ejdx).astype(jnp.float32)
    base = jnp.dot(pc, strict_lt, preferred_element_type=jnp.float32)
    base2 = base + cnt1
    cnt_ref[...] = jnp.concatenate(
        [base, pc, jnp.zeros((1, 16), jnp.float32)], axis=1)

    a1_ref[...] = a1
    a2_ref[...] = a2
    cw_ref[...] = jnp.concatenate(
        [w1n, w2n, s1_real.astype(jnp.float32), s2_real.astype(jnp.float32),
         jnp.zeros((_N, 4), jnp.float32)], axis=1)

    ck = _N // 8
    rower = lax.broadcasted_iota(jnp.int32, (ck, ck), 0)
    coler = lax.broadcasted_iota(jnp.int32, (ck, ck), 1)
    tri = (coler < rower).astype(jnp.float32)

    def chunk(k, carry):
        carry1, carry2 = carry
        sl = pl.ds(k * ck, ck)
        a1c = a1_ref[sl, :]
        a2c = a2_ref[sl, :]
        cc = cw_ref[sl, :]
        w1c = cc[:, 0:1]
        w2c = cc[:, 1:2]
        s1c = cc[:, 2:3] > 0.5
        s2c = cc[:, 3:4] > 0.5
        r1c = jnp.dot(tri, a1c, preferred_element_type=jnp.float32) + carry1
        r2c = jnp.dot(tri, a2c, preferred_element_type=jnp.float32) + carry2
        d1 = jnp.sum(a1c * (base + r1c), axis=-1, keepdims=True)
        d2 = jnp.sum(a2c * (base2 + r2c), axis=-1, keepdims=True)
        d1c = jnp.where(s1c, d1, float(_ZP))
        d2c = jnp.where(s2c, d2, float(_ZP))
        dest1 = jnp.where(s1c, d1, -1.0)
        dest2 = jnp.where(s2c, d2, -1.0)
        meta_ref[sl, :] = jnp.concatenate(
            [w1c, w2c, d1c, d2c, dest1, dest2,
             jnp.zeros((ck, 2), jnp.float32)], axis=1)
        return (carry1 + jnp.sum(a1c, axis=0, keepdims=True),
                carry2 + jnp.sum(a2c, axis=0, keepdims=True))

    lax.fori_loop(0, 8, chunk, (jnp.zeros((1, _E), jnp.float32),
                                jnp.zeros((1, _E), jnp.float32)))


def _router(xf, gate_W, logit_bias, null_logit):
    return pl.pallas_call(
        _router_kernel,
        out_shape=(
            jax.ShapeDtypeStruct((_N, 8), jnp.float32),
            jax.ShapeDtypeStruct((1, 32), jnp.float32),
            jax.ShapeDtypeStruct((1, 1), jnp.float32),
        ),
        scratch_shapes=[
            pltpu.VMEM((_N, _E), jnp.float32),
            pltpu.VMEM((_N, _E), jnp.float32),
            pltpu.VMEM((_N, 8), jnp.float32),
        ],
    )(xf, gate_W.T, logit_bias.reshape(1, _E), null_logit.reshape(1, 1))


def _sc_scatter(destcat):
    mesh = plsc.VectorSubcoreMesh(core_axis_name="c", subcore_axis_name="s")

    @functools.partial(
        pl.kernel, mesh=mesh,
        compiler_params=pltpu.CompilerParams(needs_layout_passes=False),
        out_type=jax.ShapeDtypeStruct((_P,), jnp.int32),
        scratch_types=[
            pltpu.VMEM((2 * _N,), jnp.int32),
            pltpu.VMEM((_P,), jnp.int32),
        ],
    )
    def k(dest_hbm, src_out, dest_v, src_v):
        wid = lax.axis_index("s") * 2 + lax.axis_index("c")

        @pl.when(wid == 0)
        def _():
            pltpu.sync_copy(dest_hbm, dest_v)
            zi = jnp.zeros((16,), jnp.int32)

            def zbody(i, c):
                src_v[pl.ds(i * 16, 16)] = zi
                return c

            lax.fori_loop(0, _P // 16, zbody, 0)
            iot = lax.iota(jnp.int32, 16)

            def body(i, c):
                d = dest_v[pl.ds(i * 16, 16)]
                msk = d >= 0
                ds = jnp.where(msk, d, 0)
                tok = (i % 128) * 16 + iot
                plsc.store_scatter(src_v, [ds], tok, mask=msk)
                return c

            lax.fori_loop(0, (2 * _N) // 16, body, 0)
            pltpu.sync_copy(src_v, src_out)

    return k(destcat)


def _gmm_kernel(emap_ref, tval_ref, srcr_ref, xb_ref, wg_ref, wu_ref, wd_ref,
                out_ref, wgb_ref, wub_ref, wdb_ref):
    j = pl.program_id(0)
    fresh = jnp.logical_or(j == 0, emap_ref[j] != emap_ref[jnp.maximum(j - 1, 0)])

    @pl.when(jnp.logical_and(tval_ref[j] == 1, fresh))
    def _():
        wgb_ref[...] = wg_ref[0].astype(jnp.bfloat16)
        wub_ref[...] = wu_ref[0].astype(jnp.bfloat16)
        wdb_ref[...] = wd_ref[0].astype(jnp.bfloat16)

    @pl.when(tval_ref[j] == 1)
    def _():
        sv = srcr_ref[0, 0, :][:, None]
        tok = lax.broadcasted_iota(jnp.int32, (_TG, _N), 1)
        eq = (sv == tok).astype(jnp.bfloat16)
        xs = jnp.dot(eq, xb_ref[...],
                     preferred_element_type=jnp.float32).astype(jnp.bfloat16)
        g = jnp.dot(xs, wgb_ref[...], preferred_element_type=jnp.float32)
        u = jnp.dot(xs, wub_ref[...], preferred_element_type=jnp.float32)
        h = (g * jax.nn.sigmoid(g) * u).astype(jnp.bfloat16)
        y = jnp.dot(h, wdb_ref[...], preferred_element_type=jnp.float32)
        out_ref[...] = y.astype(jnp.bfloat16)

    @pl.when(tval_ref[j] == 0)
    def _():
        out_ref[...] = jnp.zeros((_TG, _D), jnp.bfloat16)


def _grouped_mm(xb, srcr, W_gate, W_up, W_down, emap, tval):
    return pl.pallas_call(
        _gmm_kernel,
        grid_spec=pltpu.PrefetchScalarGridSpec(
            num_scalar_prefetch=2,
            grid=(_NTILE,),
            in_specs=[
                pl.BlockSpec((1, 1, _TG), lambda j, em, tv: (j, 0, 0)),
                pl.BlockSpec((_N, _D), lambda j, em, tv: (0, 0)),
                pl.BlockSpec((1, _D, _H), lambda j, em, tv: (em[j], 0, 0)),
                pl.BlockSpec((1, _D, _H), lambda j, em, tv: (em[j], 0, 0)),
                pl.BlockSpec((1, _H, _D), lambda j, em, tv: (em[j], 0, 0)),
            ],
            out_specs=pl.BlockSpec((_TG, _D), lambda j, em, tv: (j, 0)),
            scratch_shapes=[
                pltpu.VMEM((_D, _H), jnp.bfloat16),
                pltpu.VMEM((_D, _H), jnp.bfloat16),
                pltpu.VMEM((_H, _D), jnp.bfloat16),
            ],
        ),
        out_shape=jax.ShapeDtypeStruct((_P, _D), jnp.bfloat16),
        compiler_params=pltpu.CompilerParams(
            dimension_semantics=("arbitrary",),
        ),
    )(emap, tval, srcr, xb, W_gate, W_up, W_down)


def _shared_kernel(xb_ref, meta_ref, ys_ref, sg_ref, su_ref, sd_ref, out_ref,
                   sgb_ref, sub_ref, sdb_ref):
    j = pl.program_id(0)

    @pl.when(j == 0)
    def _():
        sgb_ref[...] = sg_ref[...].astype(jnp.bfloat16)
        sub_ref[...] = su_ref[...].astype(jnp.bfloat16)
        sdb_ref[...] = sd_ref[...].astype(jnp.bfloat16)

    xb = xb_ref[...]
    dn = (((1,), (1,)), ((), ()))
    g = lax.dot_general(xb, sgb_ref[...], dn, preferred_element_type=jnp.float32)
    u = lax.dot_general(xb, sub_ref[...], dn, preferred_element_type=jnp.float32)
    h = (g * jax.nn.sigmoid(g) * u).astype(jnp.bfloat16)
    sh = lax.dot_general(h, sdb_ref[...], dn, preferred_element_type=jnp.float32)

    mt = meta_ref[...]
    posr = lax.broadcasted_iota(jnp.int32, (mt.shape[0], _P), 1)
    di1 = mt[:, 2:3].astype(jnp.int32)
    di2 = mt[:, 3:4].astype(jnp.int32)
    cmb = (jnp.where(di1 == posr, mt[:, 0:1], 0.0)
           + jnp.where(di2 == posr, mt[:, 1:2], 0.0)).astype(jnp.bfloat16)
    moe = jnp.dot(cmb, ys_ref[...], preferred_element_type=jnp.float32)
    out_ref[...] = sh + moe


def _shared_combine(xb, meta, ys, sg_W, su_W, sd_W):
    tt = 256
    nt = _N // tt
    return pl.pallas_call(
        _shared_kernel,
        grid=(nt,),
        in_specs=[
            pl.BlockSpec((tt, _D), lambda j: (j, 0)),
            pl.BlockSpec((tt, 8), lambda j: (j, 0)),
            pl.BlockSpec((_P, _D), lambda j: (0, 0)),
            pl.BlockSpec((_H, _D), lambda j: (0, 0)),
            pl.BlockSpec((_H, _D), lambda j: (0, 0)),
            pl.BlockSpec((_D, _H), lambda j: (0, 0)),
        ],
        out_specs=pl.BlockSpec((tt, _D), lambda j: (j, 0)),
        out_shape=jax.ShapeDtypeStruct((_N, _D), jnp.float32),
        scratch_shapes=[
            pltpu.VMEM((_H, _D), jnp.bfloat16),
            pltpu.VMEM((_H, _D), jnp.bfloat16),
            pltpu.VMEM((_D, _H), jnp.bfloat16),
        ],
        compiler_params=pltpu.CompilerParams(
            dimension_semantics=("arbitrary",),
        ),
    )(xb, meta, ys, sg_W, su_W, sd_W)


def kernel(x, gate_W, logit_bias, null_logit, W_gate, W_up, W_down, sg_W, su_W, sd_W):
    b, t, d = x.shape
    xf = x.reshape(_N, _D)

    meta, cnts, aux = _router(xf, gate_W, logit_bias, null_logit)

    base = cnts[0, :_E]
    pc = cnts[0, _E:2 * _E]
    ends = base + pc
    jpos = jnp.arange(_NTILE, dtype=jnp.float32) * _TG
    emap = jnp.minimum(
        jnp.sum((jpos[:, None] >= ends[None, :]).astype(jnp.int32), axis=1),
        _E - 1).astype(jnp.int32)
    tval = (jpos < jnp.sum(pc)).astype(jnp.int32)

    destcat = jnp.concatenate([meta[:, 4], meta[:, 5]]).astype(jnp.int32)

    src = _sc_scatter(destcat)

    xb = xf.astype(jnp.bfloat16)
    ys = _grouped_mm(xb, src.reshape(_NTILE, 1, _TG), W_gate, W_up, W_down,
                     emap, tval)
    out = _shared_combine(xb, meta, ys, sg_W, su_W, sd_W)

    return out.reshape(b, t, d), aux[0, 0]


# speedup vs baseline: 3.7358x; 1.0507x over previous
import functools

import jax
import jax.numpy as jnp
from jax import lax
from jax.experimental import pallas as pl
from jax.experimental.pallas import tpu as pltpu
from jax.experimental.pallas import tpu_sc as plsc

_E = 8
_D = 1024
_H = 1024
_RHO = 0.5
_N = 2048
_TG = 256
_NTILE = 24
_P = _NTILE * _TG
_ZP = _P - 1


def _router_kernel(x_ref, gwt_ref, bias_ref, vnull_ref,
                   meta_ref, cnt_ref, aux_ref, a1_ref, a2_ref, cw_ref):
    x = x_ref[...]
    gwt = gwt_ref[...]
    l = jnp.dot(x, gwt, preferred_element_type=jnp.float32) + bias_ref[...]
    v = vnull_ref[0, 0]

    idx = lax.broadcasted_iota(jnp.int32, (_N, _E), 1)
    l1 = jnp.max(l, axis=-1, keepdims=True)
    i1 = jnp.min(jnp.where(l == l1, idx, _E), axis=-1, keepdims=True)
    oh1 = idx == i1
    s1_real = l1 >= v
    lm = jnp.where(oh1, -jnp.inf, l)
    l2 = jnp.max(lm, axis=-1, keepdims=True)
    i2 = jnp.min(jnp.where(lm == l2, idx, _E), axis=-1, keepdims=True)
    oh2 = idx == i2
    s2_real = s1_real & (l2 >= v)

    m = jnp.maximum(l1, v)
    el = jnp.exp(l - m)
    ev = jnp.exp(v - m)
    z = jnp.sum(el, axis=-1, keepdims=True) + 8.0 * ev
    p = el / z
    w1 = jnp.where(s1_real, jnp.sum(jnp.where(oh1, p, 0.0), axis=-1, keepdims=True), 0.0)
    w2 = jnp.where(s2_real, jnp.sum(jnp.where(oh2, p, 0.0), axis=-1, keepdims=True), 0.0)
    wsum = jnp.maximum(w1 + w2, 1e-6)
    w1n = w1 / wsum
    w2n = w2 / wsum

    elr = jnp.exp(l - l1)
    pr = elr / jnp.sum(elr, axis=-1, keepdims=True)
    p_real = jnp.mean(pr, axis=0)
    a1 = (oh1 & s1_real).astype(jnp.float32)
    a2 = (oh2 & s2_real).astype(jnp.float32)
    counts = jnp.sum(a1 + a2, axis=0)
    total = jnp.maximum(jnp.sum(counts), 1e-6)
    l_bal = _E * jnp.sum((counts / total) * p_real)
    n_real = jnp.sum(a1) + jnp.sum(a2)
    null_rate = (2.0 * _N - n_real) / (2.0 * _N)
    l_null = (null_rate - _RHO) ** 2
    lse = m + jnp.log(z)
    l_z = jnp.mean(lse * lse)
    aux = 0.02 * l_bal + 0.001 * l_z + 0.01 * l_null
    aux_ref[...] = jnp.reshape(aux, (1, 1))

    cnt1 = jnp.sum(a1, axis=0, keepdims=True)
    cnt2 = jnp.sum(a2, axis=0, keepdims=True)
    cnt = cnt1 + cnt2
    pc = jnp.ceil(cnt * (1.0 / _TG)) * _TG
    eidx = lax.broadcasted_iota(jnp.int32, (_E, _E), 0)
    ejdx = lax.broadcasted_iota(jnp.int32, (_E, _E), 1)
    strict_lt = (eidx < ejdx).astype(jnp.float32)
    base = jnp.dot(pc, strict_lt, preferred_element_type=jnp.float32)
    base2 = base + cnt1
    cnt_ref[...] = jnp.concatenate(
        [base, pc, jnp.zeros((1, 16), jnp.float32)], axis=1)

    a1_ref[...] = a1
    a2_ref[...] = a2
    cw_ref[...] = jnp.concatenate(
        [w1n, w2n, s1_real.astype(jnp.float32), s2_real.astype(jnp.float32),
         jnp.zeros((_N, 4), jnp.float32)], axis=1)

    ck = _N // 8
    rower = lax.broadcasted_iota(jnp.int32, (ck, ck), 0)
    coler = lax.broadcasted_iota(jnp.int32, (ck, ck), 1)
    tri = (coler < rower).astype(jnp.float32)

    def chunk(k, carry):
        carry1, carry2 = carry
        sl = pl.ds(k * ck, ck)
        a1c = a1_ref[sl, :]
        a2c = a2_ref[sl, :]
        cc = cw_ref[sl, :]
        w1c = cc[:, 0:1]
        w2c = cc[:, 1:2]
        s1c = cc[:, 2:3] > 0.5
        s2c = cc[:, 3:4] > 0.5
        r1c = jnp.dot(tri, a1c, preferred_element_type=jnp.float32) + carry1
        r2c = jnp.dot(tri, a2c, preferred_element_type=jnp.float32) + carry2
        d1 = jnp.sum(a1c * (base + r1c), axis=-1, keepdims=True)
        d2 = jnp.sum(a2c * (base2 + r2c), axis=-1, keepdims=True)
        d1c = jnp.where(s1c, d1, float(_ZP))
        d2c = jnp.where(s2c, d2, float(_ZP))
        dest1 = jnp.where(s1c, d1, -1.0)
        dest2 = jnp.where(s2c, d2, -1.0)
        meta_ref[sl, :] = jnp.concatenate(
            [w1c, w2c, d1c, d2c, dest1, dest2,
             jnp.zeros((ck, 2), jnp.float32)], axis=1)
        return (carry1 + jnp.sum(a1c, axis=0, keepdims=True),
                carry2 + jnp.sum(a2c, axis=0, keepdims=True))

    lax.fori_loop(0, 8, chunk, (jnp.zeros((1, _E), jnp.float32),
                                jnp.zeros((1, _E), jnp.float32)))


def _router(xf, gate_W, logit_bias, null_logit):
    return pl.pallas_call(
        _router_kernel,
        out_shape=(
            jax.ShapeDtypeStruct((_N, 8), jnp.float32),
            jax.ShapeDtypeStruct((1, 32), jnp.float32),
            jax.ShapeDtypeStruct((1, 1), jnp.float32),
        ),
        scratch_shapes=[
            pltpu.VMEM((_N, _E), jnp.float32),
            pltpu.VMEM((_N, _E), jnp.float32),
            pltpu.VMEM((_N, 8), jnp.float32),
        ],
    )(xf, gate_W.T, logit_bias.reshape(1, _E), null_logit.reshape(1, 1))


def _sc_scatter(destcat):
    mesh = plsc.VectorSubcoreMesh(core_axis_name="c", subcore_axis_name="s")

    @functools.partial(
        pl.kernel, mesh=mesh,
        compiler_params=pltpu.CompilerParams(needs_layout_passes=False),
        out_type=jax.ShapeDtypeStruct((_P,), jnp.int32),
        scratch_types=[
            pltpu.VMEM((2 * _N,), jnp.int32),
            pltpu.VMEM((_P,), jnp.int32),
        ],
    )
    def k(dest_hbm, src_out, dest_v, src_v):
        wid = lax.axis_index("s") * 2 + lax.axis_index("c")

        @pl.when(wid == 0)
        def _():
            pltpu.sync_copy(dest_hbm, dest_v)
            zi = jnp.zeros((16,), jnp.int32)

            def zbody(i, c):
                src_v[pl.ds(i * 16, 16)] = zi
                return c

            lax.fori_loop(0, _P // 16, zbody, 0)
            iot = lax.iota(jnp.int32, 16)

            def body(i, c):
                d = dest_v[pl.ds(i * 16, 16)]
                msk = d >= 0
                ds = jnp.where(msk, d, 0)
                tok = (i % 128) * 16 + iot
                plsc.store_scatter(src_v, [ds], tok, mask=msk)
                return c

            lax.fori_loop(0, (2 * _N) // 16, body, 0)
            pltpu.sync_copy(src_v, src_out)

    return k(destcat)


def _gmm_kernel(emap_ref, tval_ref, srcr_ref, xb_ref, wg_ref, wu_ref, wd_ref,
                out_ref, wgb_ref, wub_ref, wdb_ref):
    j = pl.program_id(0)
    fresh = jnp.logical_or(j == 0, emap_ref[j] != emap_ref[jnp.maximum(j - 1, 0)])

    @pl.when(jnp.logical_and(tval_ref[j] == 1, fresh))
    def _():
        wgb_ref[...] = wg_ref[0].astype(jnp.bfloat16)
        wub_ref[...] = wu_ref[0].astype(jnp.bfloat16)
        wdb_ref[...] = wd_ref[0].astype(jnp.bfloat16)

    @pl.when(tval_ref[j] == 1)
    def _():
        sv = srcr_ref[0, 0, :][:, None]
        tok = lax.broadcasted_iota(jnp.int32, (_TG, _N), 1)
        eq = (sv == tok).astype(jnp.bfloat16)
        xs = jnp.dot(eq, xb_ref[...],
                     preferred_element_type=jnp.float32).astype(jnp.bfloat16)
        g = jnp.dot(xs, wgb_ref[...], preferred_element_type=jnp.float32)
        u = jnp.dot(xs, wub_ref[...], preferred_element_type=jnp.float32)
        h = (g * jax.nn.sigmoid(g) * u).astype(jnp.bfloat16)
        y = jnp.dot(h, wdb_ref[...], preferred_element_type=jnp.float32)
        out_ref[...] = y.astype(jnp.bfloat16)

    @pl.when(tval_ref[j] == 0)
    def _():
        out_ref[...] = jnp.zeros((_TG, _D), jnp.bfloat16)


def _grouped_mm(xb, srcr, W_gate, W_up, W_down, emap, tval):
    return pl.pallas_call(
        _gmm_kernel,
        grid_spec=pltpu.PrefetchScalarGridSpec(
            num_scalar_prefetch=2,
            grid=(_NTILE,),
            in_specs=[
                pl.BlockSpec((1, 1, _TG), lambda j, em, tv: (j, 0, 0)),
                pl.BlockSpec((_N, _D), lambda j, em, tv: (0, 0)),
                pl.BlockSpec((1, _D, _H), lambda j, em, tv: (em[j], 0, 0)),
                pl.BlockSpec((1, _D, _H), lambda j, em, tv: (em[j], 0, 0)),
                pl.BlockSpec((1, _H, _D), lambda j, em, tv: (em[j], 0, 0)),
            ],
            out_specs=pl.BlockSpec((_TG, _D), lambda j, em, tv: (j, 0)),
            scratch_shapes=[
                pltpu.VMEM((_D, _H), jnp.bfloat16),
                pltpu.VMEM((_D, _H), jnp.bfloat16),
                pltpu.VMEM((_H, _D), jnp.bfloat16),
            ],
        ),
        out_shape=jax.ShapeDtypeStruct((_P, _D), jnp.bfloat16),
        compiler_params=pltpu.CompilerParams(
            dimension_semantics=("arbitrary",),
        ),
    )(emap, tval, srcr, xb, W_gate, W_up, W_down)


def _shared_kernel(xb_ref, meta_ref, ys_ref, sg_ref, su_ref, sd_ref, out_ref,
                   sgb_ref, sub_ref, sdb_ref):
    j = pl.program_id(0)

    @pl.when(j == 0)
    def _():
        sgb_ref[...] = sg_ref[...].astype(jnp.bfloat16)
        sub_ref[...] = su_ref[...].astype(jnp.bfloat16)
        sdb_ref[...] = sd_ref[...].astype(jnp.bfloat16)

    xb = xb_ref[...]
    dn = (((1,), (1,)), ((), ()))
    g = lax.dot_general(xb, sgb_ref[...], dn, preferred_element_type=jnp.float32)
    u = lax.dot_general(xb, sub_ref[...], dn, preferred_element_type=jnp.float32)
    h = (g * jax.nn.sigmoid(g) * u).astype(jnp.bfloat16)
    sh = lax.dot_general(h, sdb_ref[...], dn, preferred_element_type=jnp.float32)

    mt = meta_ref[...]
    posr = lax.broadcasted_iota(jnp.int32, (mt.shape[0], _P), 1)
    di1 = mt[:, 2:3].astype(jnp.int32)
    di2 = mt[:, 3:4].astype(jnp.int32)
    cmb = (jnp.where(di1 == posr, mt[:, 0:1], 0.0)
           + jnp.where(di2 == posr, mt[:, 1:2], 0.0)).astype(jnp.bfloat16)
    moe = jnp.dot(cmb, ys_ref[...], preferred_element_type=jnp.float32)
    out_ref[...] = sh + moe


def _shared_combine(xb, meta, ys, sg_W, su_W, sd_W):
    tt = 256
    nt = _N // tt
    return pl.pallas_call(
        _shared_kernel,
        grid=(nt,),
        in_specs=[
            pl.BlockSpec((tt, _D), lambda j: (j, 0)),
            pl.BlockSpec((tt, 8), lambda j: (j, 0)),
            pl.BlockSpec((_P, _D), lambda j: (0, 0)),
            pl.BlockSpec((_H, _D), lambda j: (0, 0)),
            pl.BlockSpec((_H, _D), lambda j: (0, 0)),
            pl.BlockSpec((_D, _H), lambda j: (0, 0)),
        ],
        out_specs=pl.BlockSpec((tt, _D), lambda j: (j, 0)),
        out_shape=jax.ShapeDtypeStruct((_N, _D), jnp.float32),
        scratch_shapes=[
            pltpu.VMEM((_H, _D), jnp.bfloat16),
            pltpu.VMEM((_H, _D), jnp.bfloat16),
            pltpu.VMEM((_D, _H), jnp.bfloat16),
        ],
        compiler_params=pltpu.CompilerParams(
            dimension_semantics=("arbitrary",),
        ),
    )(xb, meta, ys, sg_W, su_W, sd_W)


def kernel(x, gate_W, logit_bias, null_logit, W_gate, W_up, W_down, sg_W, su_W, sd_W):
    b, t, d = x.shape
    xf = x.reshape(_N, _D)

    meta, cnts, aux = _router(xf, gate_W, logit_bias, null_logit)

    base = cnts[0, :_E]
    pc = cnts[0, _E:2 * _E]
    ends = base + pc
    jpos = jnp.arange(_NTILE, dtype=jnp.float32) * _TG
    emap = jnp.minimum(
        jnp.sum((jpos[:, None] >= ends[None, :]).astype(jnp.int32), axis=1),
        _E - 1).astype(jnp.int32)
    tval = (jpos < jnp.sum(pc)).astype(jnp.int32)

    destcat = jnp.concatenate([meta[:, 4], meta[:, 5]]).astype(jnp.int32)

    src = _sc_scatter(destcat)

    xb = xf.astype(jnp.bfloat16)
    ys = _grouped_mm(xb, src.reshape(_NTILE, 1, _TG), W_gate, W_up, W_down,
                     emap, tval)
    out = _shared_combine(xb, meta, ys, sg_W, su_W, sd_W)

    return out.reshape(b, t, d), aux[0, 0]


# speedup vs baseline: 3.9507x; 1.0575x over previous
import functools

import jax
import jax.numpy as jnp
from jax import lax
from jax.experimental import pallas as pl
from jax.experimental.pallas import tpu as pltpu
from jax.experimental.pallas import tpu_sc as plsc

_E = 8
_D = 1024
_H = 1024
_RHO = 0.5
_N = 2048
_TG = 256
_NTILE = 24
_P = _NTILE * _TG
_ZP = _P - 1


def _router_kernel(x_ref, gwt_ref, bias_ref, vnull_ref,
                   meta_ref, cnt_ref, aux_ref, a1_ref, a2_ref, cw_ref):
    x = x_ref[...]
    gwt = gwt_ref[...]
    l = jnp.dot(x, gwt, preferred_element_type=jnp.float32) + bias_ref[...]
    v = vnull_ref[0, 0]

    idx = lax.broadcasted_iota(jnp.int32, (_N, _E), 1)
    l1 = jnp.max(l, axis=-1, keepdims=True)
    i1 = jnp.min(jnp.where(l == l1, idx, _E), axis=-1, keepdims=True)
    oh1 = idx == i1
    s1_real = l1 >= v
    lm = jnp.where(oh1, -jnp.inf, l)
    l2 = jnp.max(lm, axis=-1, keepdims=True)
    i2 = jnp.min(jnp.where(lm == l2, idx, _E), axis=-1, keepdims=True)
    oh2 = idx == i2
    s2_real = s1_real & (l2 >= v)

    m = jnp.maximum(l1, v)
    el = jnp.exp(l - m)
    ev = jnp.exp(v - m)
    z = jnp.sum(el, axis=-1, keepdims=True) + 8.0 * ev
    p = el / z
    w1 = jnp.where(s1_real, jnp.sum(jnp.where(oh1, p, 0.0), axis=-1, keepdims=True), 0.0)
    w2 = jnp.where(s2_real, jnp.sum(jnp.where(oh2, p, 0.0), axis=-1, keepdims=True), 0.0)
    wsum = jnp.maximum(w1 + w2, 1e-6)
    w1n = w1 / wsum
    w2n = w2 / wsum

    elr = jnp.exp(l - l1)
    pr = elr / jnp.sum(elr, axis=-1, keepdims=True)
    p_real = jnp.mean(pr, axis=0)
    a1 = (oh1 & s1_real).astype(jnp.float32)
    a2 = (oh2 & s2_real).astype(jnp.float32)
    counts = jnp.sum(a1 + a2, axis=0)
    total = jnp.maximum(jnp.sum(counts), 1e-6)
    l_bal = _E * jnp.sum((counts / total) * p_real)
    n_real = jnp.sum(a1) + jnp.sum(a2)
    null_rate = (2.0 * _N - n_real) / (2.0 * _N)
    l_null = (null_rate - _RHO) ** 2
    lse = m + jnp.log(z)
    l_z = jnp.mean(lse * lse)
    aux = 0.02 * l_bal + 0.001 * l_z + 0.01 * l_null
    aux_ref[...] = jnp.reshape(aux, (1, 1))

    cnt1 = jnp.sum(a1, axis=0, keepdims=True)
    cnt2 = jnp.sum(a2, axis=0, keepdims=True)
    cnt = cnt1 + cnt2
    pc = jnp.ceil(cnt * (1.0 / _TG)) * _TG
    eidx = lax.broadcasted_iota(jnp.int32, (_E, _E), 0)
    ejdx = lax.broadcasted_iota(jnp.int32, (_E, _E), 1)
    strict_lt = (eidx < ejdx).astype(jnp.float32)
    base = jnp.dot(pc, strict_lt, preferred_element_type=jnp.float32)
    base2 = base + cnt1
    cnt_ref[...] = jnp.concatenate(
        [base, pc, jnp.zeros((1, 16), jnp.float32)], axis=1)

    a1_ref[...] = a1
    a2_ref[...] = a2
    cw_ref[...] = jnp.concatenate(
        [w1n, w2n, s1_real.astype(jnp.float32), s2_real.astype(jnp.float32),
         jnp.zeros((_N, 4), jnp.float32)], axis=1)

    ck = _N // 8
    rower = lax.broadcasted_iota(jnp.int32, (ck, ck), 0)
    coler = lax.broadcasted_iota(jnp.int32, (ck, ck), 1)
    tri = (coler < rower).astype(jnp.float32)

    def chunk(k, carry):
        carry1, carry2 = carry
        sl = pl.ds(k * ck, ck)
        a1c = a1_ref[sl, :]
        a2c = a2_ref[sl, :]
        cc = cw_ref[sl, :]
        w1c = cc[:, 0:1]
        w2c = cc[:, 1:2]
        s1c = cc[:, 2:3] > 0.5
        s2c = cc[:, 3:4] > 0.5
        r1c = jnp.dot(tri, a1c, preferred_element_type=jnp.float32) + carry1
        r2c = jnp.dot(tri, a2c, preferred_element_type=jnp.float32) + carry2
        d1 = jnp.sum(a1c * (base + r1c), axis=-1, keepdims=True)
        d2 = jnp.sum(a2c * (base2 + r2c), axis=-1, keepdims=True)
        d1c = jnp.where(s1c, d1, float(_ZP))
        d2c = jnp.where(s2c, d2, float(_ZP))
        dest1 = jnp.where(s1c, d1, -1.0)
        dest2 = jnp.where(s2c, d2, -1.0)
        meta_ref[sl, :] = jnp.concatenate(
            [w1c, w2c, d1c, d2c, dest1, dest2,
             jnp.zeros((ck, 2), jnp.float32)], axis=1)
        return (carry1 + jnp.sum(a1c, axis=0, keepdims=True),
                carry2 + jnp.sum(a2c, axis=0, keepdims=True))

    lax.fori_loop(0, 8, chunk, (jnp.zeros((1, _E), jnp.float32),
                                jnp.zeros((1, _E), jnp.float32)))


def _router(xf, gate_W, logit_bias, null_logit):
    return pl.pallas_call(
        _router_kernel,
        out_shape=(
            jax.ShapeDtypeStruct((_N, 8), jnp.float32),
            jax.ShapeDtypeStruct((1, 32), jnp.float32),
            jax.ShapeDtypeStruct((1, 1), jnp.float32),
        ),
        scratch_shapes=[
            pltpu.VMEM((_N, _E), jnp.float32),
            pltpu.VMEM((_N, _E), jnp.float32),
            pltpu.VMEM((_N, 8), jnp.float32),
        ],
    )(xf, gate_W.T, logit_bias.reshape(1, _E), null_logit.reshape(1, 1))


def _sc_scatter(destcat):
    mesh = plsc.VectorSubcoreMesh(core_axis_name="c", subcore_axis_name="s")

    @functools.partial(
        pl.kernel, mesh=mesh,
        compiler_params=pltpu.CompilerParams(needs_layout_passes=False),
        out_type=jax.ShapeDtypeStruct((_P,), jnp.int32),
        scratch_types=[
            pltpu.VMEM((2 * _N,), jnp.int32),
            pltpu.VMEM((_P,), jnp.int32),
        ],
    )
    def k(dest_hbm, src_out, dest_v, src_v):
        wid = lax.axis_index("s") * 2 + lax.axis_index("c")

        @pl.when(wid == 0)
        def _():
            pltpu.sync_copy(dest_hbm, dest_v)
            zi = jnp.zeros((16,), jnp.int32)

            def zbody(i, c):
                src_v[pl.ds(i * 16, 16)] = zi
                return c

            lax.fori_loop(0, _P // 16, zbody, 0)
            iot = lax.iota(jnp.int32, 16)

            def body(i, c):
                d = dest_v[pl.ds(i * 16, 16)]
                msk = d >= 0
                ds = jnp.where(msk, d, 0)
                tok = (i % 128) * 16 + iot
                plsc.store_scatter(src_v, [ds], tok, mask=msk)
                return c

            lax.fori_loop(0, (2 * _N) // 16, body, 0)
            pltpu.sync_copy(src_v, src_out)

    return k(destcat)


def _fused_kernel(emap_ref, tval_ref, srcr_ref, xb_ref, wg_ref, wu_ref,
                  wd_ref, meta_ref, sg_ref, su_ref, sd_ref, out_ref,
                  ys_ref, wgb_ref, wub_ref):
    j = pl.program_id(0)
    is_g = j < _NTILE
    fresh = jnp.logical_or(j == 0, emap_ref[j] != emap_ref[jnp.maximum(j - 1, 0)])

    @pl.when(jnp.logical_and(is_g, jnp.logical_and(tval_ref[j] == 1, fresh)))
    def _():
        wgb_ref[...] = wg_ref[0].astype(jnp.bfloat16)
        wub_ref[...] = wu_ref[0].astype(jnp.bfloat16)

    @pl.when(jnp.logical_and(is_g, tval_ref[j] == 1))
    def _():
        sv = srcr_ref[0, 0, :][:, None]
        tok = lax.broadcasted_iota(jnp.int32, (_TG, _N), 1)
        eq = (sv == tok).astype(jnp.bfloat16)
        xs = jnp.dot(eq, xb_ref[...],
                     preferred_element_type=jnp.float32).astype(jnp.bfloat16)
        g = jnp.dot(xs, wgb_ref[...], preferred_element_type=jnp.float32)
        u = jnp.dot(xs, wub_ref[...], preferred_element_type=jnp.float32)
        h = (g * jax.nn.sigmoid(g) * u).astype(jnp.bfloat16)
        y = jnp.dot(h, wd_ref[0].astype(jnp.bfloat16),
                    preferred_element_type=jnp.float32)
        ys_ref[pl.ds(j * _TG, _TG), :] = y.astype(jnp.bfloat16)

    @pl.when(jnp.logical_and(is_g, tval_ref[j] == 0))
    def _():
        ys_ref[pl.ds(j * _TG, _TG), :] = jnp.zeros((_TG, _D), jnp.bfloat16)

    @pl.when(jnp.logical_not(is_g))
    def _():
        xb = xb_ref[pl.ds((j - _NTILE) * 256, 256), :]
        dn = (((1,), (1,)), ((), ()))
        g = lax.dot_general(xb, sg_ref[...].astype(jnp.bfloat16), dn,
                            preferred_element_type=jnp.float32)
        u = lax.dot_general(xb, su_ref[...].astype(jnp.bfloat16), dn,
                            preferred_element_type=jnp.float32)
        h = (g * jax.nn.sigmoid(g) * u).astype(jnp.bfloat16)
        sh = lax.dot_general(h, sd_ref[...].astype(jnp.bfloat16), dn,
                             preferred_element_type=jnp.float32)

        mt = meta_ref[...]
        posr = lax.broadcasted_iota(jnp.int32, (256, _P), 1)
        di1 = mt[:, 2:3].astype(jnp.int32)
        di2 = mt[:, 3:4].astype(jnp.int32)
        cmb = (jnp.where(di1 == posr, mt[:, 0:1], 0.0)
               + jnp.where(di2 == posr, mt[:, 1:2], 0.0)).astype(jnp.bfloat16)
        moe = jnp.dot(cmb, ys_ref[...], preferred_element_type=jnp.float32)
        out_ref[...] = sh + moe


def _fused(xb, srcr, W_gate, W_up, W_down, meta, sg_W, su_W, sd_W, emap, tval):
    nt2 = _N // 256
    return pl.pallas_call(
        _fused_kernel,
        grid_spec=pltpu.PrefetchScalarGridSpec(
            num_scalar_prefetch=2,
            grid=(_NTILE + nt2,),
            in_specs=[
                pl.BlockSpec((1, 1, _TG),
                             lambda j, em, tv: (jnp.minimum(j, _NTILE - 1), 0, 0)),
                pl.BlockSpec((_N, _D), lambda j, em, tv: (0, 0)),
                pl.BlockSpec((1, _D, _H), lambda j, em, tv: (em[j], 0, 0)),
                pl.BlockSpec((1, _D, _H), lambda j, em, tv: (em[j], 0, 0)),
                pl.BlockSpec((1, _H, _D), lambda j, em, tv: (em[j], 0, 0)),
                pl.BlockSpec((256, 8),
                             lambda j, em, tv: (jnp.maximum(j - _NTILE, 0), 0)),
                pl.BlockSpec((_H, _D), lambda j, em, tv: (0, 0)),
                pl.BlockSpec((_H, _D), lambda j, em, tv: (0, 0)),
                pl.BlockSpec((_D, _H), lambda j, em, tv: (0, 0)),
            ],
            out_specs=pl.BlockSpec((256, _D),
                                   lambda j, em, tv: (jnp.maximum(j - _NTILE, 0), 0)),
            scratch_shapes=[
                pltpu.VMEM((_P, _D), jnp.bfloat16),
                pltpu.VMEM((_D, _H), jnp.bfloat16),
                pltpu.VMEM((_D, _H), jnp.bfloat16),
            ],
        ),
        out_shape=jax.ShapeDtypeStruct((_N, _D), jnp.float32),
        compiler_params=pltpu.CompilerParams(
            dimension_semantics=("arbitrary",),
            vmem_limit_bytes=100 * 1024 * 1024,
        ),
    )(emap, tval, srcr, xb, W_gate, W_up, W_down, meta, sg_W, su_W, sd_W)


def kernel(x, gate_W, logit_bias, null_logit, W_gate, W_up, W_down, sg_W, su_W, sd_W):
    b, t, d = x.shape
    xf = x.reshape(_N, _D)

    meta, cnts, aux = _router(xf, gate_W, logit_bias, null_logit)

    base = cnts[0, :_E]
    pc = cnts[0, _E:2 * _E]
    ends = base + pc
    jpos = jnp.arange(_NTILE, dtype=jnp.float32) * _TG
    emap = jnp.minimum(
        jnp.sum((jpos[:, None] >= ends[None, :]).astype(jnp.int32), axis=1),
        _E - 1).astype(jnp.int32)
    tval = (jpos < jnp.sum(pc)).astype(jnp.int32)

    destcat = jnp.concatenate([meta[:, 4], meta[:, 5]]).astype(jnp.int32)

    src = _sc_scatter(destcat)

    xb = xf.astype(jnp.bfloat16)
    nt2 = _N // 256
    emap2 = jnp.concatenate([emap, jnp.full((nt2,), _E - 1, jnp.int32)])
    tval2 = jnp.concatenate([tval, jnp.zeros((nt2,), jnp.int32)])
    out = _fused(xb, src.reshape(_NTILE, 1, _TG), W_gate, W_up, W_down,
                 meta, sg_W, su_W, sd_W, emap2, tval2)

    return out.reshape(b, t, d), aux[0, 0]


# speedup vs baseline: 3.9886x; 1.0096x over previous
import functools

import jax
import jax.numpy as jnp
from jax import lax
from jax.experimental import pallas as pl
from jax.experimental.pallas import tpu as pltpu
from jax.experimental.pallas import tpu_sc as plsc

_E = 8
_D = 1024
_H = 1024
_RHO = 0.5
_N = 2048
_TG = 256
_NTILE = 24
_P = _NTILE * _TG
_ZP = _P - 1


def _router_kernel(x_ref, gwt_ref, bias_ref, vnull_ref,
                   meta_ref, cnt_ref, aux_ref, xb_ref, a1_ref, a2_ref, cw_ref):
    x = x_ref[...]
    xb_ref[...] = x.astype(jnp.bfloat16)
    gwt = gwt_ref[...]
    lfull = jnp.dot(x, gwt, preferred_element_type=jnp.float32)
    l = lfull[:, :_E] + bias_ref[...]
    v = vnull_ref[0, 0]

    idx = lax.broadcasted_iota(jnp.int32, (_N, _E), 1)
    l1 = jnp.max(l, axis=-1, keepdims=True)
    i1 = jnp.min(jnp.where(l == l1, idx, _E), axis=-1, keepdims=True)
    oh1 = idx == i1
    s1_real = l1 >= v
    lm = jnp.where(oh1, -jnp.inf, l)
    l2 = jnp.max(lm, axis=-1, keepdims=True)
    i2 = jnp.min(jnp.where(lm == l2, idx, _E), axis=-1, keepdims=True)
    oh2 = idx == i2
    s2_real = s1_real & (l2 >= v)

    m = jnp.maximum(l1, v)
    el = jnp.exp(l - m)
    ev = jnp.exp(v - m)
    z = jnp.sum(el, axis=-1, keepdims=True) + 8.0 * ev
    p = el / z
    w1 = jnp.where(s1_real, jnp.sum(jnp.where(oh1, p, 0.0), axis=-1, keepdims=True), 0.0)
    w2 = jnp.where(s2_real, jnp.sum(jnp.where(oh2, p, 0.0), axis=-1, keepdims=True), 0.0)
    wsum = jnp.maximum(w1 + w2, 1e-6)
    w1n = w1 / wsum
    w2n = w2 / wsum

    elr = jnp.exp(l - l1)
    pr = elr / jnp.sum(elr, axis=-1, keepdims=True)
    p_real = jnp.mean(pr, axis=0)
    a1 = (oh1 & s1_real).astype(jnp.float32)
    a2 = (oh2 & s2_real).astype(jnp.float32)
    counts = jnp.sum(a1 + a2, axis=0)
    total = jnp.maximum(jnp.sum(counts), 1e-6)
    l_bal = _E * jnp.sum((counts / total) * p_real)
    n_real = jnp.sum(a1) + jnp.sum(a2)
    null_rate = (2.0 * _N - n_real) / (2.0 * _N)
    l_null = (null_rate - _RHO) ** 2
    lse = m + jnp.log(z)
    l_z = jnp.mean(lse * lse)
    aux = 0.02 * l_bal + 0.001 * l_z + 0.01 * l_null
    aux_ref[...] = jnp.reshape(aux, (1, 1))

    cnt1 = jnp.sum(a1, axis=0, keepdims=True)
    cnt2 = jnp.sum(a2, axis=0, keepdims=True)
    cnt = cnt1 + cnt2
    pc = jnp.ceil(cnt * (1.0 / _TG)) * _TG
    eidx = lax.broadcasted_iota(jnp.int32, (_E, _E), 0)
    ejdx = lax.broadcasted_iota(jnp.int32, (_E, _E), 1)
    strict_lt = (eidx < ejdx).astype(jnp.float32)
    base = jnp.dot(pc, strict_lt, preferred_element_type=jnp.float32)
    base2 = base + cnt1
    cnt_ref[...] = jnp.concatenate(
        [base, pc, jnp.zeros((1, 16), jnp.float32)], axis=1)

    a1_ref[...] = a1
    a2_ref[...] = a2
    cw_ref[...] = jnp.concatenate(
        [w1n, w2n, s1_real.astype(jnp.float32), s2_real.astype(jnp.float32),
         jnp.zeros((_N, 4), jnp.float32)], axis=1)

    ck = _N // 8
    rower = lax.broadcasted_iota(jnp.int32, (ck, ck), 0)
    coler = lax.broadcasted_iota(jnp.int32, (ck, ck), 1)
    tri = (coler < rower).astype(jnp.float32)

    def chunk(k, carry):
        carry1, carry2 = carry
        sl = pl.ds(k * ck, ck)
        a1c = a1_ref[sl, :]
        a2c = a2_ref[sl, :]
        cc = cw_ref[sl, :]
        w1c = cc[:, 0:1]
        w2c = cc[:, 1:2]
        s1c = cc[:, 2:3] > 0.5
        s2c = cc[:, 3:4] > 0.5
        r1c = jnp.dot(tri, a1c, preferred_element_type=jnp.float32) + carry1
        r2c = jnp.dot(tri, a2c, preferred_element_type=jnp.float32) + carry2
        d1 = jnp.sum(a1c * (base + r1c), axis=-1, keepdims=True)
        d2 = jnp.sum(a2c * (base2 + r2c), axis=-1, keepdims=True)
        d1c = jnp.where(s1c, d1, float(_ZP))
        d2c = jnp.where(s2c, d2, float(_ZP))
        dest1 = jnp.where(s1c, d1, -1.0)
        dest2 = jnp.where(s2c, d2, -1.0)
        meta_ref[sl, :] = jnp.concatenate(
            [w1c, w2c, d1c, d2c, dest1, dest2,
             jnp.zeros((ck, 2), jnp.float32)], axis=1)
        return (carry1 + jnp.sum(a1c, axis=0, keepdims=True),
                carry2 + jnp.sum(a2c, axis=0, keepdims=True))

    lax.fori_loop(0, 8, chunk, (jnp.zeros((1, _E), jnp.float32),
                                jnp.zeros((1, _E), jnp.float32)))


def _router(xf, gate_W, logit_bias, null_logit):
    return pl.pallas_call(
        _router_kernel,
        out_shape=(
            jax.ShapeDtypeStruct((_N, 8), jnp.float32),
            jax.ShapeDtypeStruct((1, 32), jnp.float32),
            jax.ShapeDtypeStruct((1, 1), jnp.float32),
            jax.ShapeDtypeStruct((_N, _D), jnp.bfloat16),
        ),
        scratch_shapes=[
            pltpu.VMEM((_N, _E), jnp.float32),
            pltpu.VMEM((_N, _E), jnp.float32),
            pltpu.VMEM((_N, 8), jnp.float32),
        ],
    )(xf, jnp.zeros((_D, 128), xf.dtype).at[:, :_E].set(gate_W.T),
      logit_bias.reshape(1, _E), null_logit.reshape(1, 1))


def _sc_scatter(destcat):
    mesh = plsc.VectorSubcoreMesh(core_axis_name="c", subcore_axis_name="s")

    @functools.partial(
        pl.kernel, mesh=mesh,
        compiler_params=pltpu.CompilerParams(needs_layout_passes=False),
        out_type=jax.ShapeDtypeStruct((_P,), jnp.int32),
        scratch_types=[
            pltpu.VMEM((2 * _N,), jnp.int32),
            pltpu.VMEM((_P,), jnp.int32),
        ],
    )
    def k(dest_hbm, src_out, dest_v, src_v):
        wid = lax.axis_index("s") * 2 + lax.axis_index("c")

        @pl.when(wid == 0)
        def _():
            pltpu.sync_copy(dest_hbm, dest_v)
            zi = jnp.zeros((16,), jnp.int32)

            def zbody(i, c):
                src_v[pl.ds(i * 16, 16)] = zi
                return c

            lax.fori_loop(0, _P // 16, zbody, 0)
            iot = lax.iota(jnp.int32, 16)

            def body(i, c):
                d = dest_v[pl.ds(i * 16, 16)]
                msk = d >= 0
                ds = jnp.where(msk, d, 0)
                tok = (i % 128) * 16 + iot
                plsc.store_scatter(src_v, [ds], tok, mask=msk)
                return c

            lax.fori_loop(0, (2 * _N) // 16, body, 0)
            pltpu.sync_copy(src_v, src_out)

    return k(destcat)


def _fused_kernel(emap_ref, tval_ref, srcr_ref, xb_ref, wg_ref, wu_ref,
                  wd_ref, meta_ref, sg_ref, su_ref, sd_ref, out_ref,
                  ys_ref, wgb_ref, wub_ref):
    j = pl.program_id(0)
    is_g = j < _NTILE
    fresh = jnp.logical_or(j == 0, emap_ref[j] != emap_ref[jnp.maximum(j - 1, 0)])

    @pl.when(jnp.logical_and(is_g, jnp.logical_and(tval_ref[j] == 1, fresh)))
    def _():
        wgb_ref[...] = wg_ref[0].astype(jnp.bfloat16)
        wub_ref[...] = wu_ref[0].astype(jnp.bfloat16)

    @pl.when(jnp.logical_and(is_g, tval_ref[j] == 1))
    def _():
        sv = srcr_ref[0, 0, :][:, None]
        tok = lax.broadcasted_iota(jnp.int32, (_TG, _N), 1)
        eq = (sv == tok).astype(jnp.bfloat16)
        xs = jnp.dot(eq, xb_ref[...],
                     preferred_element_type=jnp.float32).astype(jnp.bfloat16)
        g = jnp.dot(xs, wgb_ref[...], preferred_element_type=jnp.float32)
        u = jnp.dot(xs, wub_ref[...], preferred_element_type=jnp.float32)
        h = (g * jax.nn.sigmoid(g) * u).astype(jnp.bfloat16)
        y = jnp.dot(h, wd_ref[0].astype(jnp.bfloat16),
                    preferred_element_type=jnp.float32)
        ys_ref[pl.ds(j * _TG, _TG), :] = y.astype(jnp.bfloat16)

    @pl.when(jnp.logical_and(is_g, tval_ref[j] == 0))
    def _():
        ys_ref[pl.ds(j * _TG, _TG), :] = jnp.zeros((_TG, _D), jnp.bfloat16)

    @pl.when(jnp.logical_not(is_g))
    def _():
        xb = xb_ref[pl.ds((j - _NTILE) * 256, 256), :]
        dn = (((1,), (1,)), ((), ()))
        g = lax.dot_general(xb, sg_ref[...].astype(jnp.bfloat16), dn,
                            preferred_element_type=jnp.float32)
        u = lax.dot_general(xb, su_ref[...].astype(jnp.bfloat16), dn,
                            preferred_element_type=jnp.float32)
        h = (g * jax.nn.sigmoid(g) * u).astype(jnp.bfloat16)
        sh = lax.dot_general(h, sd_ref[...].astype(jnp.bfloat16), dn,
                             preferred_element_type=jnp.float32)

        mt = meta_ref[...]
        posr = lax.broadcasted_iota(jnp.int32, (256, _P), 1)
        di1 = mt[:, 2:3].astype(jnp.int32)
        di2 = mt[:, 3:4].astype(jnp.int32)
        cmb = (jnp.where(di1 == posr, mt[:, 0:1], 0.0)
               + jnp.where(di2 == posr, mt[:, 1:2], 0.0)).astype(jnp.bfloat16)
        moe = jnp.dot(cmb, ys_ref[...], preferred_element_type=jnp.float32)
        out_ref[...] = sh + moe


def _fused(xb, srcr, W_gate, W_up, W_down, meta, sg_W, su_W, sd_W, emap, tval):
    nt2 = _N // 256
    return pl.pallas_call(
        _fused_kernel,
        grid_spec=pltpu.PrefetchScalarGridSpec(
            num_scalar_prefetch=2,
            grid=(_NTILE + nt2,),
            in_specs=[
                pl.BlockSpec((1, 1, _TG),
                             lambda j, em, tv: (jnp.minimum(j, _NTILE - 1), 0, 0)),
                pl.BlockSpec((_N, _D), lambda j, em, tv: (0, 0)),
                pl.BlockSpec((1, _D, _H), lambda j, em, tv: (em[j], 0, 0)),
                pl.BlockSpec((1, _D, _H), lambda j, em, tv: (em[j], 0, 0)),
                pl.BlockSpec((1, _H, _D), lambda j, em, tv: (em[j], 0, 0)),
                pl.BlockSpec((256, 8),
                             lambda j, em, tv: (jnp.maximum(j - _NTILE, 0), 0)),
                pl.BlockSpec((_H, _D), lambda j, em, tv: (0, 0)),
                pl.BlockSpec((_H, _D), lambda j, em, tv: (0, 0)),
                pl.BlockSpec((_D, _H), lambda j, em, tv: (0, 0)),
            ],
            out_specs=pl.BlockSpec((256, _D),
                                   lambda j, em, tv: (jnp.maximum(j - _NTILE, 0), 0)),
            scratch_shapes=[
                pltpu.VMEM((_P, _D), jnp.bfloat16),
                pltpu.VMEM((_D, _H), jnp.bfloat16),
                pltpu.VMEM((_D, _H), jnp.bfloat16),
            ],
        ),
        out_shape=jax.ShapeDtypeStruct((_N, _D), jnp.float32),
        compiler_params=pltpu.CompilerParams(
            dimension_semantics=("arbitrary",),
            vmem_limit_bytes=100 * 1024 * 1024,
        ),
    )(emap, tval, srcr, xb, W_gate, W_up, W_down, meta, sg_W, su_W, sd_W)


def kernel(x, gate_W, logit_bias, null_logit, W_gate, W_up, W_down, sg_W, su_W, sd_W):
    b, t, d = x.shape
    xf = x.reshape(_N, _D)

    meta, cnts, aux, xb = _router(xf, gate_W, logit_bias, null_logit)

    base = cnts[0, :_E]
    pc = cnts[0, _E:2 * _E]
    ends = base + pc
    jpos = jnp.arange(_NTILE, dtype=jnp.float32) * _TG
    emap = jnp.minimum(
        jnp.sum((jpos[:, None] >= ends[None, :]).astype(jnp.int32), axis=1),
        _E - 1).astype(jnp.int32)
    tval = (jpos < jnp.sum(pc)).astype(jnp.int32)

    destcat = jnp.concatenate([meta[:, 4], meta[:, 5]]).astype(jnp.int32)

    src = _sc_scatter(destcat)

    nt2 = _N // 256
    emap2 = jnp.concatenate([emap, jnp.full((nt2,), _E - 1, jnp.int32)])
    tval2 = jnp.concatenate([tval, jnp.zeros((nt2,), jnp.int32)])
    out = _fused(xb, src.reshape(_NTILE, 1, _TG), W_gate, W_up, W_down,
                 meta, sg_W, su_W, sd_W, emap2, tval2)

    return out.reshape(b, t, d), aux[0, 0]


# speedup vs baseline: 3.9922x; 1.0009x over previous
import functools

import jax
import jax.numpy as jnp
from jax import lax
from jax.experimental import pallas as pl
from jax.experimental.pallas import tpu as pltpu
from jax.experimental.pallas import tpu_sc as plsc

_E = 8
_D = 1024
_H = 1024
_RHO = 0.5
_N = 2048
_TG = 256
_NTILE = 24
_P = _NTILE * _TG
_ZP = _P - 1


def _router_kernel(x_ref, gwt_ref, bias_ref, vnull_ref,
                   meta_ref, cnt_ref, aux_ref, xb_ref, a1_ref, a2_ref, cw_ref):
    x = x_ref[...]
    xb_ref[...] = x.astype(jnp.bfloat16)
    gwt = gwt_ref[...]
    lfull = jnp.dot(x, gwt, preferred_element_type=jnp.float32)
    l = lfull[:, :_E] + bias_ref[...]
    v = vnull_ref[0, 0]

    idx = lax.broadcasted_iota(jnp.int32, (_N, _E), 1)
    l1 = jnp.max(l, axis=-1, keepdims=True)
    i1 = jnp.min(jnp.where(l == l1, idx, _E), axis=-1, keepdims=True)
    oh1 = idx == i1
    s1_real = l1 >= v
    lm = jnp.where(oh1, -jnp.inf, l)
    l2 = jnp.max(lm, axis=-1, keepdims=True)
    i2 = jnp.min(jnp.where(lm == l2, idx, _E), axis=-1, keepdims=True)
    oh2 = idx == i2
    s2_real = s1_real & (l2 >= v)

    m = jnp.maximum(l1, v)
    el = jnp.exp(l - m)
    ev = jnp.exp(v - m)
    z = jnp.sum(el, axis=-1, keepdims=True) + 8.0 * ev
    p = el / z
    w1 = jnp.where(s1_real, jnp.sum(jnp.where(oh1, p, 0.0), axis=-1, keepdims=True), 0.0)
    w2 = jnp.where(s2_real, jnp.sum(jnp.where(oh2, p, 0.0), axis=-1, keepdims=True), 0.0)
    wsum = jnp.maximum(w1 + w2, 1e-6)
    w1n = w1 / wsum
    w2n = w2 / wsum

    elr = jnp.exp(l - l1)
    pr = elr / jnp.sum(elr, axis=-1, keepdims=True)
    p_real = jnp.mean(pr, axis=0)
    a1 = (oh1 & s1_real).astype(jnp.float32)
    a2 = (oh2 & s2_real).astype(jnp.float32)
    counts = jnp.sum(a1 + a2, axis=0)
    total = jnp.maximum(jnp.sum(counts), 1e-6)
    l_bal = _E * jnp.sum((counts / total) * p_real)
    n_real = jnp.sum(a1) + jnp.sum(a2)
    null_rate = (2.0 * _N - n_real) / (2.0 * _N)
    l_null = (null_rate - _RHO) ** 2
    lse = m + jnp.log(z)
    l_z = jnp.mean(lse * lse)
    aux = 0.02 * l_bal + 0.001 * l_z + 0.01 * l_null
    aux_ref[...] = jnp.reshape(aux, (1, 1))

    cnt1 = jnp.sum(a1, axis=0, keepdims=True)
    cnt2 = jnp.sum(a2, axis=0, keepdims=True)
    cnt = cnt1 + cnt2
    pc = jnp.ceil(cnt * (1.0 / _TG)) * _TG
    eidx = lax.broadcasted_iota(jnp.int32, (_E, _E), 0)
    ejdx = lax.broadcasted_iota(jnp.int32, (_E, _E), 1)
    strict_lt = (eidx < ejdx).astype(jnp.float32)
    base = jnp.dot(pc, strict_lt, preferred_element_type=jnp.float32)
    base2 = base + cnt1
    cnt_ref[...] = jnp.concatenate(
        [base, pc, jnp.zeros((1, 16), jnp.float32)], axis=1)

    a1_ref[...] = a1
    a2_ref[...] = a2
    cw_ref[...] = jnp.concatenate(
        [w1n, w2n, s1_real.astype(jnp.float32), s2_real.astype(jnp.float32),
         jnp.zeros((_N, 4), jnp.float32)], axis=1)

    ck = _N // 8
    rower = lax.broadcasted_iota(jnp.int32, (ck, ck), 0)
    coler = lax.broadcasted_iota(jnp.int32, (ck, ck), 1)
    tri = (coler < rower).astype(jnp.float32)

    def chunk(k, carry):
        carry1, carry2 = carry
        sl = pl.ds(k * ck, ck)
        a1c = a1_ref[sl, :]
        a2c = a2_ref[sl, :]
        cc = cw_ref[sl, :]
        w1c = cc[:, 0:1]
        w2c = cc[:, 1:2]
        s1c = cc[:, 2:3] > 0.5
        s2c = cc[:, 3:4] > 0.5
        r1c = jnp.dot(tri, a1c, preferred_element_type=jnp.float32) + carry1
        r2c = jnp.dot(tri, a2c, preferred_element_type=jnp.float32) + carry2
        d1 = jnp.sum(a1c * (base + r1c), axis=-1, keepdims=True)
        d2 = jnp.sum(a2c * (base2 + r2c), axis=-1, keepdims=True)
        d1c = jnp.where(s1c, d1, float(_ZP))
        d2c = jnp.where(s2c, d2, float(_ZP))
        dest1 = jnp.where(s1c, d1, -1.0)
        dest2 = jnp.where(s2c, d2, -1.0)
        meta_ref[sl, :] = jnp.concatenate(
            [w1c, w2c, d1c, d2c, dest1, dest2,
             jnp.zeros((ck, 2), jnp.float32)], axis=1)
        return (carry1 + jnp.sum(a1c, axis=0, keepdims=True),
                carry2 + jnp.sum(a2c, axis=0, keepdims=True))

    lax.fori_loop(0, 8, chunk, (jnp.zeros((1, _E), jnp.float32),
                                jnp.zeros((1, _E), jnp.float32)))


def _router(xf, gate_W, logit_bias, null_logit):
    return pl.pallas_call(
        _router_kernel,
        out_shape=(
            jax.ShapeDtypeStruct((_N, 8), jnp.float32),
            jax.ShapeDtypeStruct((1, 32), jnp.float32),
            jax.ShapeDtypeStruct((1, 1), jnp.float32),
            jax.ShapeDtypeStruct((_N, _D), jnp.bfloat16),
        ),
        scratch_shapes=[
            pltpu.VMEM((_N, _E), jnp.float32),
            pltpu.VMEM((_N, _E), jnp.float32),
            pltpu.VMEM((_N, 8), jnp.float32),
        ],
    )(xf, jnp.zeros((_D, 128), xf.dtype).at[:, :_E].set(gate_W.T),
      logit_bias.reshape(1, _E), null_logit.reshape(1, 1))


def _sc_scatter(destcat):
    mesh = plsc.VectorSubcoreMesh(core_axis_name="c", subcore_axis_name="s")

    @functools.partial(
        pl.kernel, mesh=mesh,
        compiler_params=pltpu.CompilerParams(needs_layout_passes=False),
        out_type=jax.ShapeDtypeStruct((_P,), jnp.int32),
        scratch_types=[
            pltpu.VMEM((2 * _N,), jnp.int32),
            pltpu.VMEM((_P,), jnp.int32),
        ],
    )
    def k(dest_hbm, src_out, dest_v, src_v):
        wid = lax.axis_index("s") * 2 + lax.axis_index("c")

        @pl.when(wid == 0)
        def _():
            pltpu.sync_copy(dest_hbm, dest_v)
            zi = jnp.zeros((16,), jnp.int32)

            def zbody(i, c):
                src_v[pl.ds(i * 16, 16)] = zi
                return c

            lax.fori_loop(0, _P // 16, zbody, 0)
            iot = lax.iota(jnp.int32, 16)

            def body(i, c):
                d = dest_v[pl.ds(i * 16, 16)]
                msk = d >= 0
                ds = jnp.where(msk, d, 0)
                tok = (i % 128) * 16 + iot
                plsc.store_scatter(src_v, [ds], tok, mask=msk)
                return c

            lax.fori_loop(0, (2 * _N) // 16, body, 0)
            pltpu.sync_copy(src_v, src_out)

    return k(destcat)


def _fused_kernel(emap_ref, tval_ref, srcr_ref, xb_ref, wg_ref, wu_ref,
                  wd_ref, meta_ref, sg_ref, su_ref, sd_ref, out_ref,
                  ys_ref, wgb_ref, wub_ref):
    j = pl.program_id(0)
    is_g = j < _NTILE
    fresh = jnp.logical_or(j == 0, emap_ref[j] != emap_ref[jnp.maximum(j - 1, 0)])

    @pl.when(jnp.logical_and(is_g, jnp.logical_and(tval_ref[j] == 1, fresh)))
    def _():
        wgb_ref[...] = wg_ref[0].astype(jnp.bfloat16)
        wub_ref[...] = wu_ref[0].astype(jnp.bfloat16)

    @pl.when(jnp.logical_and(is_g, tval_ref[j] == 1))
    def _():
        sv = srcr_ref[0, 0, :][:, None]
        tok = lax.broadcasted_iota(jnp.int32, (_TG, _N), 1)
        eq = (sv == tok).astype(jnp.bfloat16)
        xs = jnp.dot(eq, xb_ref[...],
                     preferred_element_type=jnp.float32).astype(jnp.bfloat16)
        g = jnp.dot(xs, wgb_ref[...], preferred_element_type=jnp.float32)
        u = jnp.dot(xs, wub_ref[...], preferred_element_type=jnp.float32)
        h = (g * jax.nn.sigmoid(g) * u).astype(jnp.bfloat16)
        y = jnp.dot(h, wd_ref[0].astype(jnp.bfloat16),
                    preferred_element_type=jnp.float32)
        ys_ref[pl.ds(j * _TG, _TG), :] = y.astype(jnp.bfloat16)

    @pl.when(jnp.logical_and(is_g, tval_ref[j] == 0))
    def _():
        ys_ref[pl.ds(j * _TG, _TG), :] = jnp.zeros((_TG, _D), jnp.bfloat16)

    @pl.when(jnp.logical_not(is_g))
    def _():
        xb = xb_ref[pl.ds((j - _NTILE) * 256, 256), :]
        dn = (((1,), (1,)), ((), ()))
        g = lax.dot_general(xb, sg_ref[...].astype(jnp.bfloat16), dn,
                            preferred_element_type=jnp.float32)
        u = lax.dot_general(xb, su_ref[...].astype(jnp.bfloat16), dn,
                            preferred_element_type=jnp.float32)
        h = (g * jax.nn.sigmoid(g) * u).astype(jnp.bfloat16)
        sh = lax.dot_general(h, sd_ref[...].astype(jnp.bfloat16), dn,
                             preferred_element_type=jnp.float32)

        mt = meta_ref[...]
        posr = lax.broadcasted_iota(jnp.int32, (256, _P), 1)
        di1 = mt[:, 2:3].astype(jnp.int32)
        di2 = mt[:, 3:4].astype(jnp.int32)
        cmb = (jnp.where(di1 == posr, mt[:, 0:1], 0.0)
               + jnp.where(di2 == posr, mt[:, 1:2], 0.0)).astype(jnp.bfloat16)
        moe = jnp.dot(cmb, ys_ref[...], preferred_element_type=jnp.float32)
        out_ref[...] = sh + moe


def _fused(xb, srcr, W_gate, W_up, W_down, meta, sg_W, su_W, sd_W, emap, tval):
    nt2 = _N // 256
    return pl.pallas_call(
        _fused_kernel,
        grid_spec=pltpu.PrefetchScalarGridSpec(
            num_scalar_prefetch=2,
            grid=(_NTILE + nt2,),
            in_specs=[
                pl.BlockSpec((1, 1, _TG),
                             lambda j, em, tv: (jnp.minimum(j, _NTILE - 1), 0, 0)),
                pl.BlockSpec((_N, _D), lambda j, em, tv: (0, 0)),
                pl.BlockSpec((1, _D, _H), lambda j, em, tv: (em[j], 0, 0)),
                pl.BlockSpec((1, _D, _H), lambda j, em, tv: (em[j], 0, 0)),
                pl.BlockSpec((1, _H, _D), lambda j, em, tv: (em[j], 0, 0)),
                pl.BlockSpec((256, 8),
                             lambda j, em, tv: (jnp.maximum(j - _NTILE, 0), 0)),
                pl.BlockSpec((_H, _D), lambda j, em, tv: (0, 0)),
                pl.BlockSpec((_H, _D), lambda j, em, tv: (0, 0)),
                pl.BlockSpec((_D, _H), lambda j, em, tv: (0, 0)),
            ],
            out_specs=pl.BlockSpec((256, _D),
                                   lambda j, em, tv: (jnp.maximum(j - _NTILE, 0), 0)),
            scratch_shapes=[
                pltpu.VMEM((_P, _D), jnp.bfloat16),
                pltpu.VMEM((_D, _H), jnp.bfloat16),
                pltpu.VMEM((_D, _H), jnp.bfloat16),
            ],
        ),
        out_shape=jax.ShapeDtypeStruct((_N, _D), jnp.float32),
        compiler_params=pltpu.CompilerParams(
            dimension_semantics=("arbitrary",),
            vmem_limit_bytes=100 * 1024 * 1024,
            allow_input_fusion=[True] * 11,
        ),
    )(emap, tval, srcr, xb, W_gate, W_up, W_down, meta, sg_W, su_W, sd_W)


def kernel(x, gate_W, logit_bias, null_logit, W_gate, W_up, W_down, sg_W, su_W, sd_W):
    b, t, d = x.shape
    xf = x.reshape(_N, _D)

    meta, cnts, aux, xb = _router(xf, gate_W, logit_bias, null_logit)

    base = cnts[0, :_E]
    pc = cnts[0, _E:2 * _E]
    ends = base + pc
    jpos = jnp.arange(_NTILE, dtype=jnp.float32) * _TG
    emap = jnp.minimum(
        jnp.sum((jpos[:, None] >= ends[None, :]).astype(jnp.int32), axis=1),
        _E - 1).astype(jnp.int32)
    tval = (jpos < jnp.sum(pc)).astype(jnp.int32)

    destcat = jnp.concatenate([meta[:, 4], meta[:, 5]]).astype(jnp.int32)

    src = _sc_scatter(destcat)

    nt2 = _N // 256
    emap2 = jnp.concatenate([emap, jnp.full((nt2,), _E - 1, jnp.int32)])
    tval2 = jnp.concatenate([tval, jnp.zeros((nt2,), jnp.int32)])
    out = _fused(xb, src.reshape(_NTILE, 1, _TG), W_gate, W_up, W_down,
                 meta, sg_W, su_W, sd_W, emap2, tval2)

    return out.reshape(b, t, d), aux[0, 0]
